# Initial kernel scaffold; baseline (speedup 1.0000x reference)
import jax
import jax.numpy as jnp
from jax.experimental import pallas as pl


def kernel(x, edge_index, edge_attr, line_edge_index, params):
    raise NotImplementedError("write your pallas kernel here")


# speedup vs baseline: 20.7121x; 20.7121x over previous
import functools

import jax
import jax.numpy as jnp
from jax import lax
from jax.experimental import pallas as pl
from jax.experimental.pallas import tpu as pltpu
from jax.experimental.pallas import tpu_sc as plsc

_CAP = 8
_HEADS = 4
_NW = 32
_BV = 400
_BVS = 80
_BE = 2000
_NEG = -1e30


def _lrelu(t, s):
    return jnp.where(t > 0, t, s * t)


@functools.cache
def _make_gather(t_rows, d, b):
    del t_rows
    bpw = b // _NW
    assert b % (_NW * 128) == 0, (b,)
    nchunks = bpw // 128
    fire = min(nchunks, 20, max(1, (400 * 1024) // (128 * d * 4)))
    while nchunks % fire:
        fire -= 1
    ngroups = nchunks // fire
    grp_rows = fire * 128
    mesh = plsc.VectorSubcoreMesh(core_axis_name="c", subcore_axis_name="s")

    @functools.partial(
        pl.kernel,
        mesh=mesh,
        compiler_params=pltpu.CompilerParams(use_tc_tiling_on_sc=False),
        out_type=jax.ShapeDtypeStruct((b, d), jnp.float32),
        scratch_types=[
            pltpu.VMEM((bpw,), jnp.int32),
            pltpu.VMEM((grp_rows, d), jnp.float32),
            pltpu.SemaphoreType.DMA,
        ],
    )
    def gk(table_hbm, idx_hbm, out_hbm, idx_v, rows_v, sem):
        wid = lax.axis_index("s") * 2 + lax.axis_index("c")
        base = wid * bpw
        pltpu.sync_copy(idx_hbm.at[pl.ds(base, bpw)], idx_v)

        def group(gi, carry):
            row0 = gi * grp_rows
            handles = [
                pltpu.async_copy(
                    table_hbm.at[idx_v.at[pl.ds(row0 + f * 128, 128)]],
                    rows_v.at[pl.ds(f * 128, 128)],
                    sem,
                )
                for f in range(fire)
            ]
            for h in handles:
                h.wait()
            pltpu.sync_copy(rows_v, out_hbm.at[pl.ds(base + row0, grp_rows)])
            return carry

        if ngroups == 1:
            group(0, 0)
        else:
            lax.fori_loop(0, ngroups, group, 0)

    return gk


def _sc_gather(table, idx):
    return _make_gather(table.shape[0], table.shape[1], idx.shape[0])(table, idx)


def _full(shape):
    return pl.BlockSpec(shape, lambda i: (0,) * len(shape))


def _blk(shape):
    return pl.BlockSpec(shape, lambda i: (i,) + (0,) * (len(shape) - 1))


def _rep8(a):
    bv, c = a.shape
    return jnp.broadcast_to(a[:, None, :], (bv, _CAP, c)).reshape(bv * _CAP, c)


def _p0n_body(x_ref, wle, aee, wl2e, w6e, b6e, km_ref, out_ref):
    x = x_ref[...]
    pe = jnp.dot(x, wle[...], preferred_element_type=jnp.float32)
    aeh = (pe.reshape(_BV, _HEADS, 16) * aee[...][None]).sum(-1)
    ee2 = jnp.dot(x, wl2e[...], preferred_element_type=jnp.float32)
    xe6 = jnp.dot(x, w6e[...], preferred_element_type=jnp.float32) + b6e[...]
    pad = jnp.zeros((_BV, 4), jnp.float32)
    out_ref[...] = jnp.concatenate([aeh, ee2, xe6, km_ref[...], pad], axis=1)


def _p0e_body(ea_ref, w1, asrc, adst, b1, epki_ref, e1_ref, h1d_ref, epf_ref):
    ea = ea_ref[...]
    xl = jnp.dot(ea, w1[...], preferred_element_type=jnp.float32)
    xlh = xl.reshape(_BE, _HEADS, 16)
    a_s = (xlh * asrc[...][None]).sum(-1)
    a_d = (xlh * adst[...][None]).sum(-1)
    e1_ref[...] = jnp.concatenate(
        [a_s, a_d, xl, jnp.zeros((_BE, 8), jnp.float32)], axis=1)
    h1d_ref[...] = xlh.mean(axis=1) + b1[...]
    cnt = epki_ref[:, 0:1]
    mf = epki_ref[:, 1:2]
    dis = jnp.where(cnt > 0, lax.rsqrt(jnp.maximum(cnt, 1.0)), 0.0)
    epf_ref[...] = jnp.concatenate(
        [dis, mf, jnp.zeros((_BE, 6), jnp.float32)], axis=1)


def _t1_body(gi_ref, go_ref, ng_ref, b1, r1_ref):
    rows = _BVS * _CAP
    gi = gi_ref[...]
    go = go_ref[...]
    ain = gi[:, 0:4].reshape(_BVS, _CAP, 4)
    xin = gi[:, 8:72].reshape(_BVS, _CAP, 64)
    aso = go[:, 0:4]
    ado = go[:, 4:8]
    xo = go[:, 8:72]
    ae_r = _rep8(ng_ref[:, 0:4])
    km_r = _rep8(ng_ref[:, 36:44])

    l_list = []
    mx = None
    for k in range(_CAP):
        lk = _lrelu(_rep8(ain[:, k, :]) + ado + ae_r, 0.2)
        lk = jnp.where(km_r[:, k:k + 1] > 0, lk, _NEG)
        l_list.append(lk)
        mx = lk if mx is None else jnp.maximum(mx, lk)
    ls = _lrelu(aso + ado + ae_r, 0.2)
    mx = jnp.maximum(mx, ls)
    es = jnp.exp(ls - mx)
    ssum = es
    acc = jnp.broadcast_to(es[:, :, None], (rows, 4, 16)).reshape(rows, 64) * xo
    for k in range(_CAP):
        ek = jnp.exp(l_list[k] - mx) * km_r[:, k:k + 1]
        ssum = ssum + ek
        ekb = jnp.broadcast_to(ek[:, :, None], (rows, 4, 16)).reshape(rows, 64)
        acc = acc + ekb * _rep8(xin[:, k, :])
    out = acc / jnp.broadcast_to(
        (ssum + 1e-16)[:, :, None], (rows, 4, 16)).reshape(rows, 64)
    r1_ref[...] = out.reshape(rows, _HEADS, 16).mean(axis=1) + b1[...]


def _t2_body(rg1_ref, h1d_ref, epf_ref, wl, bl, wr, br, b2, e2_ref, h2d_ref):
    mf = epf_ref[:, 1:2]
    h1 = jnp.where(mf > 0, rg1_ref[...], h1d_ref[...])
    xl2 = jnp.dot(h1, wl[...], preferred_element_type=jnp.float32) + bl[...]
    xr2 = jnp.dot(h1, wr[...], preferred_element_type=jnp.float32) + br[...]
    e2_ref[...] = jnp.concatenate([xl2, xr2], axis=1)
    h2d_ref[...] = xl2 + b2[...]


def _t3_body(gi_ref, go_ref, ng_ref, att, b2, r2_ref):
    xl2i = gi_ref[:, 0:16].reshape(_BVS, _CAP, 16)
    xl2o = go_ref[:, 0:16]
    xr2o = go_ref[:, 16:32]
    ee2_r = _rep8(ng_ref[:, 4:20])
    km_r = _rep8(ng_ref[:, 36:44])
    at = att[...]

    l_list = []
    x_list = []
    mx = None
    for k in range(_CAP):
        xk = _rep8(xl2i[:, k, :])
        x_list.append(xk)
        zk = _lrelu(xk + xr2o + ee2_r, 0.2)
        lk = (zk * at).sum(axis=1, keepdims=True)
        lk = jnp.where(km_r[:, k:k + 1] > 0, lk, _NEG)
        l_list.append(lk)
        mx = lk if mx is None else jnp.maximum(mx, lk)
    zs = _lrelu(xl2o + xr2o + ee2_r, 0.2)
    ls = (zs * at).sum(axis=1, keepdims=True)
    mx = jnp.maximum(mx, ls)
    es = jnp.exp(ls - mx)
    ssum = es
    acc = es * xl2o
    for k in range(_CAP):
        ek = jnp.exp(l_list[k] - mx) * km_r[:, k:k + 1]
        ssum = ssum + ek
        acc = acc + ek * x_list[k]
    r2_ref[...] = acc / (ssum + 1e-16) + b2[...]


def _t4_body(rg2_ref, h2d_ref, epf_ref, w4, v4, b4, e3_ref, hv4_ref):
    mf = epf_ref[:, 1:2]
    dis = epf_ref[:, 0:1]
    h2 = jnp.where(mf > 0, rg2_ref[...], h2d_ref[...])
    e3_ref[...] = jnp.dot(h2, w4[...], preferred_element_type=jnp.float32) * dis
    hv4_ref[...] = jnp.dot(h2, v4[...], preferred_element_type=jnp.float32) + b4[...]


def _t5_body(gi_ref, ng_ref, s_ref):
    g = gi_ref[...].reshape(_BVS, _CAP, 16)
    km = ng_ref[:, 36:44]
    s = jnp.zeros((_BVS, 16), jnp.float32)
    for k in range(_CAP):
        s = s + g[:, k, :] * km[:, k:k + 1]
    s_ref[...] = s


def _t6_body(sg_ref, hv_ref, epf_ref, w5, v5, b5, e4_ref, hv5_ref):
    dis = epf_ref[:, 0:1]
    h3 = jnp.maximum(sg_ref[...] * dis + hv_ref[...], 0.0)
    e4_ref[...] = jnp.dot(h3, w5[...], preferred_element_type=jnp.float32) * dis
    hv5_ref[...] = jnp.dot(h3, v5[...], preferred_element_type=jnp.float32) + b5[...]


def _t8_body(sg_ref, hv_ref, epf_ref, h4_ref):
    dis = epf_ref[:, 0:1]
    h4_ref[...] = jnp.maximum(sg_ref[...] * dis + hv_ref[...], 0.0)


def _t9_body(gi_ref, ng_ref, s_ref):
    g = gi_ref[...].reshape(_BVS, _CAP, 16)
    xe6 = ng_ref[:, 20:36]
    km = ng_ref[:, 36:44]
    s = jnp.zeros((_BVS, 16), jnp.float32)
    for k in range(_CAP):
        s = s + jnp.maximum(g[:, k, :] + xe6, 0.0) * km[:, k:k + 1]
    s_ref[...] = s


def _t10_body(sg_ref, h4_ref, epf_ref, eps, w1, b1, w2, b2, w3, b3, out_ref):
    mf = epf_ref[:, 1:2]
    t = (1.0 + eps[...]) * h4_ref[...] + sg_ref[...] * mf
    t = _lrelu(jnp.dot(t, w1[...], preferred_element_type=jnp.float32) + b1[...], 0.01)
    t = _lrelu(jnp.dot(t, w2[...], preferred_element_type=jnp.float32) + b2[...], 0.01)
    out_ref[...] = jnp.dot(t, w3[...], preferred_element_type=jnp.float32) + b3[...]


def _group_lists(key, n, n_e):
    order = jnp.argsort(key, stable=True).astype(jnp.int32)
    ks = key[order]
    off = jnp.searchsorted(ks, jnp.arange(n, dtype=key.dtype)).astype(jnp.int32)
    off2 = jnp.searchsorted(ks, jnp.arange(n, dtype=key.dtype), side='right')
    cnt_cap = jnp.minimum(off2.astype(jnp.int32) - off, _CAP)
    rank_sorted = jnp.arange(n_e, dtype=jnp.int32) - off[ks]
    lst = jnp.zeros((n, _CAP), jnp.int32).at[ks, rank_sorted].set(
        order, mode='drop')
    rank = jnp.zeros(n_e, jnp.int32).at[order].set(rank_sorted)
    return lst, cnt_cap, rank


def _pad_idx(idx, mult):
    b = idx.shape[0]
    bp = ((b + mult - 1) // mult) * mult
    return jnp.pad(idx, (0, bp - b))


def kernel(x, edge_index, edge_attr, line_edge_index, params):
    del line_edge_index
    p = params
    n = x.shape[0]
    n_e = edge_attr.shape[0]
    ng = n // _BV
    eg = n_e // _BE
    sg = n // _BVS
    f32 = jnp.float32

    src = edge_index[0].astype(jnp.int32)
    dst = edge_index[1].astype(jnp.int32)

    in_list, ic, _ = _group_lists(dst, n, n_e)
    _, _, rank_out = _group_lists(src, n, n_e)
    g_idx = src * _CAP + jnp.minimum(rank_out, _CAP - 1)
    has_slot = rank_out < _CAP
    m_e = has_slot & (ic[src] > 0)
    cnt_e = jnp.where(has_slot, ic[src], 0)
    out_list = jnp.zeros((n, _CAP), jnp.int32).at[src, rank_out].set(
        jnp.arange(n_e, dtype=jnp.int32), mode='drop')

    in_flat = _pad_idx(in_list.reshape(-1), _NW * 128)
    out_flat = _pad_idx(out_list.reshape(-1), _NW * 128)
    g_pad = _pad_idx(g_idx, _NW * 128)
    src_pad = _pad_idx(src, _NW * 128)

    km = (jnp.arange(_CAP)[None, :] < ic[:, None]).astype(f32)
    epki = jnp.concatenate(
        [cnt_e.astype(f32)[:, None], m_e.astype(f32)[:, None],
         jnp.zeros((n_e, 6), f32)], axis=1)

    r2 = lambda w: w.reshape(1, -1)

    node_g = pl.pallas_call(
        _p0n_body, grid=(ng,),
        in_specs=[_blk((_BV, 128)), _full((128, 64)), _full((_HEADS, 16)),
                  _full((128, 16)), _full((128, 16)), _full((1, 16)),
                  _blk((_BV, _CAP))],
        out_specs=_blk((_BV, 48)),
        out_shape=jax.ShapeDtypeStruct((n, 48), f32),
    )(x, p['g1_lin_edge'], p['g1_att_edge'], p['g2_lin_edge'], p['g6_we'],
      r2(p['g6_be']), km)

    edge1, h1d, epf = pl.pallas_call(
        _p0e_body, grid=(eg,),
        in_specs=[_blk((_BE, 16)), _full((16, 64)), _full((_HEADS, 16)),
                  _full((_HEADS, 16)), _full((1, 16)), _blk((_BE, 8))],
        out_specs=(_blk((_BE, 80)), _blk((_BE, 16)), _blk((_BE, 8))),
        out_shape=(jax.ShapeDtypeStruct((n_e, 80), f32),
                   jax.ShapeDtypeStruct((n_e, 16), f32),
                   jax.ShapeDtypeStruct((n_e, 8), f32)),
    )(edge_attr, p['g1_lin'], p['g1_att_src'], p['g1_att_dst'],
      r2(p['g1_bias']), epki)

    gi1 = _sc_gather(edge1, in_flat)
    go1 = _sc_gather(edge1, out_flat)
    r1 = pl.pallas_call(
        _t1_body, grid=(sg,),
        in_specs=[_blk((_BVS * _CAP, 80)), _blk((_BVS * _CAP, 80)),
                  _blk((_BVS, 48)), _full((1, 16))],
        out_specs=_blk((_BVS * _CAP, 16)),
        out_shape=jax.ShapeDtypeStruct((n * _CAP, 16), f32),
    )(gi1, go1, node_g, r2(p['g1_bias']))

    rg1 = _sc_gather(r1, g_pad)
    edge2, h2d = pl.pallas_call(
        _t2_body, grid=(eg,),
        in_specs=[_blk((_BE, 16)), _blk((_BE, 16)), _blk((_BE, 8)),
                  _full((16, 16)), _full((1, 16)), _full((16, 16)),
                  _full((1, 16)), _full((1, 16))],
        out_specs=(_blk((_BE, 32)), _blk((_BE, 16))),
        out_shape=(jax.ShapeDtypeStruct((n_e, 32), f32),
                   jax.ShapeDtypeStruct((n_e, 16), f32)),
    )(rg1, h1d, epf, p['g2_wl'], r2(p['g2_bl']), p['g2_wr'], r2(p['g2_br']),
      r2(p['g2_bias']))

    gi2 = _sc_gather(edge2, in_flat)
    go2 = _sc_gather(edge2, out_flat)
    r2a = pl.pallas_call(
        _t3_body, grid=(sg,),
        in_specs=[_blk((_BVS * _CAP, 32)), _blk((_BVS * _CAP, 32)),
                  _blk((_BVS, 48)), _full((1, 16)), _full((1, 16))],
        out_specs=_blk((_BVS * _CAP, 16)),
        out_shape=jax.ShapeDtypeStruct((n * _CAP, 16), f32),
    )(gi2, go2, node_g, r2(p['g2_att']), r2(p['g2_bias']))

    rg2 = _sc_gather(r2a, g_pad)
    edge3, hv4 = pl.pallas_call(
        _t4_body, grid=(eg,),
        in_specs=[_blk((_BE, 16)), _blk((_BE, 16)), _blk((_BE, 8)),
                  _full((16, 16)), _full((16, 16)), _full((1, 16))],
        out_specs=(_blk((_BE, 16)), _blk((_BE, 16))),
        out_shape=(jax.ShapeDtypeStruct((n_e, 16), f32),
                   jax.ShapeDtypeStruct((n_e, 16), f32)),
    )(rg2, h2d, epf, p['a4_w'], p['a4_v'], r2(p['a4_b']))

    gi3 = _sc_gather(edge3, in_flat)
    s1 = pl.pallas_call(
        _t5_body, grid=(sg,),
        in_specs=[_blk((_BVS * _CAP, 16)), _blk((_BVS, 48))],
        out_specs=_blk((_BVS, 16)),
        out_shape=jax.ShapeDtypeStruct((n, 16), f32),
    )(gi3, node_g)
    sg1 = _sc_gather(s1, src_pad)
    edge4, hv5 = pl.pallas_call(
        _t6_body, grid=(eg,),
        in_specs=[_blk((_BE, 16)), _blk((_BE, 16)), _blk((_BE, 8)),
                  _full((16, 16)), _full((16, 16)), _full((1, 16))],
        out_specs=(_blk((_BE, 16)), _blk((_BE, 16))),
        out_shape=(jax.ShapeDtypeStruct((n_e, 16), f32),
                   jax.ShapeDtypeStruct((n_e, 16), f32)),
    )(sg1, hv4, epf, p['a5_w'], p['a5_v'], r2(p['a5_b']))

    gi4 = _sc_gather(edge4, in_flat)
    s2 = pl.pallas_call(
        _t5_body, grid=(sg,),
        in_specs=[_blk((_BVS * _CAP, 16)), _blk((_BVS, 48))],
        out_specs=_blk((_BVS, 16)),
        out_shape=jax.ShapeDtypeStruct((n, 16), f32),
    )(gi4, node_g)
    sg2 = _sc_gather(s2, src_pad)
    h4 = pl.pallas_call(
        _t8_body, grid=(eg,),
        in_specs=[_blk((_BE, 16)), _blk((_BE, 16)), _blk((_BE, 8))],
        out_specs=_blk((_BE, 16)),
        out_shape=jax.ShapeDtypeStruct((n_e, 16), f32),
    )(sg2, hv5, epf)

    gi5 = _sc_gather(h4, in_flat)
    s6 = pl.pallas_call(
        _t9_body, grid=(sg,),
        in_specs=[_blk((_BVS * _CAP, 16)), _blk((_BVS, 48))],
        out_specs=_blk((_BVS, 16)),
        out_shape=jax.ShapeDtypeStruct((n, 16), f32),
    )(gi5, node_g)
    sg6 = _sc_gather(s6, src_pad)
    new_ea = pl.pallas_call(
        _t10_body, grid=(eg,),
        in_specs=[_blk((_BE, 16)), _blk((_BE, 16)), _blk((_BE, 8)),
                  _full((1, 1)), _full((16, 32)), _full((1, 32)),
                  _full((32, 16)), _full((1, 16)), _full((16, 16)),
                  _full((1, 16))],
        out_specs=_blk((_BE, 16)),
        out_shape=jax.ShapeDtypeStruct((n_e, 16), f32),
    )(sg6, h4, epf, p['g6_eps'].reshape(1, 1), p['g6_m_w1'], r2(p['g6_m_b1']),
      p['g6_m_w2'], r2(p['g6_m_b2']), p['g6_m_w3'], r2(p['g6_m_b3']))

    return new_ea


# speedup vs baseline: 30.6685x; 1.4807x over previous
import functools

import jax
import jax.numpy as jnp
from jax import lax
from jax.experimental import pallas as pl
from jax.experimental.pallas import tpu as pltpu
from jax.experimental.pallas import tpu_sc as plsc

_CAP = 8
_HEADS = 4
_NW = 32
_BV = 400
_BVS = 200
_BE = 2000
_NEG = -1e30


def _lrelu(t, s):
    return jnp.where(t > 0, t, s * t)


@functools.cache
def _make_gather(t_rows, d, b):
    del t_rows
    bpw = b // _NW
    assert b % (_NW * 128) == 0, (b,)
    nchunks = bpw // 128
    fire = min(nchunks, 20, max(1, (400 * 1024) // (128 * d * 4)))
    while nchunks % fire:
        fire -= 1
    ngroups = nchunks // fire
    grp_rows = fire * 128
    mesh = plsc.VectorSubcoreMesh(core_axis_name="c", subcore_axis_name="s")

    @functools.partial(
        pl.kernel,
        mesh=mesh,
        compiler_params=pltpu.CompilerParams(use_tc_tiling_on_sc=False),
        out_type=jax.ShapeDtypeStruct((b, d), jnp.float32),
        scratch_types=[
            pltpu.VMEM((bpw,), jnp.int32),
            pltpu.VMEM((grp_rows, d), jnp.float32),
            pltpu.SemaphoreType.DMA,
        ],
    )
    def gk(table_hbm, idx_hbm, out_hbm, idx_v, rows_v, sem):
        wid = lax.axis_index("s") * 2 + lax.axis_index("c")
        base = wid * bpw
        pltpu.sync_copy(idx_hbm.at[pl.ds(base, bpw)], idx_v)

        def group(gi, carry):
            row0 = gi * grp_rows
            handles = [
                pltpu.async_copy(
                    table_hbm.at[idx_v.at[pl.ds(row0 + f * 128, 128)]],
                    rows_v.at[pl.ds(f * 128, 128)],
                    sem,
                )
                for f in range(fire)
            ]
            for h in handles:
                h.wait()
            pltpu.sync_copy(rows_v, out_hbm.at[pl.ds(base + row0, grp_rows)])
            return carry

        if ngroups == 1:
            group(0, 0)
        else:
            lax.fori_loop(0, ngroups, group, 0)

    return gk


def _sc_gather(table, idx):
    return _make_gather(table.shape[0], table.shape[1], idx.shape[0])(table, idx)


def _full(shape):
    return pl.BlockSpec(shape, lambda i: (0,) * len(shape))


def _blk(shape):
    return pl.BlockSpec(shape, lambda i: (i,) + (0,) * (len(shape) - 1))


def _rep8(a):
    bv, c = a.shape
    return jnp.broadcast_to(a[:, None, :], (bv, _CAP, c)).reshape(bv * _CAP, c)


def _p0n_body(x_ref, wle, aee, wl2e, w6e, b6e, km_ref, out_ref):
    x = x_ref[...]
    pe = jnp.dot(x, wle[...], preferred_element_type=jnp.float32)
    aeh = (pe.reshape(_BV, _HEADS, 16) * aee[...][None]).sum(-1)
    ee2 = jnp.dot(x, wl2e[...], preferred_element_type=jnp.float32)
    xe6 = jnp.dot(x, w6e[...], preferred_element_type=jnp.float32) + b6e[...]
    pad = jnp.zeros((_BV, 4), jnp.float32)
    out_ref[...] = jnp.concatenate([aeh, ee2, xe6, km_ref[...], pad], axis=1)


def _p0e_body(ea_ref, w1, asrc, adst, b1, epki_ref, e1_ref, h1d_ref, epf_ref):
    ea = ea_ref[...]
    xl = jnp.dot(ea, w1[...], preferred_element_type=jnp.float32)
    xlh = xl.reshape(_BE, _HEADS, 16)
    a_s = (xlh * asrc[...][None]).sum(-1)
    a_d = (xlh * adst[...][None]).sum(-1)
    e1_ref[...] = jnp.concatenate(
        [a_s, a_d, xl, jnp.zeros((_BE, 8), jnp.float32)], axis=1)
    h1d_ref[...] = xlh.mean(axis=1) + b1[...]
    cnt = epki_ref[:, 0:1]
    mf = epki_ref[:, 1:2]
    dis = jnp.where(cnt > 0, lax.rsqrt(jnp.maximum(cnt, 1.0)), 0.0)
    epf_ref[...] = jnp.concatenate(
        [dis, mf, jnp.zeros((_BE, 6), jnp.float32)], axis=1)


def _lbc(a, c):
    return jnp.broadcast_to(a, (a.shape[0], c))


def _t1_body(gi_ref, go_ref, ng_ref, b1, r1_ref):
    rows = _BVS * _CAP
    gi = gi_ref[...]
    go = go_ref[...]
    t8 = lambda a: jnp.concatenate([a] * _CAP, axis=1)
    ain32 = jnp.concatenate(
        [gi[:, 80 * k:80 * k + 4] for k in range(_CAP)], axis=1)
    aso = go[:, 0:4]
    ado = go[:, 4:8]
    xo = go[:, 8:72]
    ae_j = _rep8(ng_ref[:, 0:4])
    km = ng_ref[:, 36:44]
    km32 = jnp.concatenate(
        [_lbc(km[:, k:k + 1], 4) for k in range(_CAP)], axis=1)
    kmh_r = _rep8(km32)

    l32 = _lrelu(_rep8(ain32) + t8(ado) + t8(ae_j), 0.2)
    l32 = jnp.where(kmh_r > 0, l32, _NEG)
    ls = _lrelu(aso + ado + ae_j, 0.2)
    mx = ls
    for k in range(_CAP):
        mx = jnp.maximum(mx, l32[:, 4 * k:4 * k + 4])
    e32 = jnp.exp(l32 - t8(mx)) * kmh_r
    es = jnp.exp(ls - mx)
    ssum = es
    for k in range(_CAP):
        ssum = ssum + e32[:, 4 * k:4 * k + 4]
    ssum = ssum + 1e-16
    b16 = lambda a: jnp.concatenate(
        [_lbc(a[:, h:h + 1], 16) for h in range(_HEADS)], axis=1)
    acc = b16(es) * xo
    for k in range(_CAP):
        acc = acc + b16(e32[:, 4 * k:4 * k + 4]) * _rep8(
            gi[:, 80 * k + 8:80 * k + 72])
    out = acc / b16(ssum)
    hsum = out[:, 0:16]
    for h in range(1, _HEADS):
        hsum = hsum + out[:, 16 * h:16 * h + 16]
    r1_ref[...] = hsum * 0.25 + b1[...]

def _t2_body(rg1_ref, h1d_ref, epf_ref, wl, bl, wr, br, b2, e2_ref, h2d_ref):
    mf = epf_ref[:, 1:2]
    h1 = jnp.where(mf > 0, rg1_ref[...], h1d_ref[...])
    xl2 = jnp.dot(h1, wl[...], preferred_element_type=jnp.float32) + bl[...]
    xr2 = jnp.dot(h1, wr[...], preferred_element_type=jnp.float32) + br[...]
    e2_ref[...] = jnp.concatenate([xl2, xr2], axis=1)
    h2d_ref[...] = xl2 + b2[...]


def _t3_body(gi_ref, go_ref, ng_ref, att, att_bd, sum_mat, b2, r2_ref):
    rows = _BVS * _CAP
    gi = gi_ref[...]
    t8 = lambda a: jnp.concatenate([a] * _CAP, axis=1)
    xl2i = jnp.concatenate(
        [gi[:, 32 * k:32 * k + 16] for k in range(_CAP)], axis=1)
    xr_all = _rep8(xl2i)
    xl2o = go_ref[:, 0:16]
    xr2o = go_ref[:, 16:32]
    ee2_r = _rep8(ng_ref[:, 4:20])
    km_r = _rep8(ng_ref[:, 36:44])
    z = _lrelu(xr_all + t8(xr2o) + t8(ee2_r), 0.2)
    lk = jnp.dot(z, att_bd[...], preferred_element_type=jnp.float32)
    lk = jnp.where(km_r > 0, lk, _NEG)
    zs = _lrelu(xl2o + xr2o + ee2_r, 0.2)
    ls = (zs * att[...]).sum(axis=1, keepdims=True)
    mx = jnp.maximum(jnp.max(lk, axis=1, keepdims=True), ls)
    e8 = jnp.exp(lk - mx) * km_r
    es = jnp.exp(ls - mx)
    ssum = e8.sum(axis=1, keepdims=True) + es + 1e-16
    e_t = jnp.concatenate(
        [_lbc(e8[:, k:k + 1], 16) for k in range(_CAP)], axis=1)
    agg = jnp.dot(xr_all * e_t, sum_mat[...],
                  preferred_element_type=jnp.float32)
    r2_ref[...] = (agg + es * xl2o) / ssum + b2[...]

def _t4_body(rg2_ref, h2d_ref, epf_ref, w4, v4, b4, e3_ref, hv4_ref):
    mf = epf_ref[:, 1:2]
    dis = epf_ref[:, 0:1]
    h2 = jnp.where(mf > 0, rg2_ref[...], h2d_ref[...])
    e3_ref[...] = jnp.dot(h2, w4[...], preferred_element_type=jnp.float32) * dis
    hv4_ref[...] = jnp.dot(h2, v4[...], preferred_element_type=jnp.float32) + b4[...]


def _t5_body(gi_ref, ng_ref, s_ref):
    gi = gi_ref[...]
    km = ng_ref[:, 36:44]
    s = jnp.zeros((_BVS, 16), jnp.float32)
    for k in range(_CAP):
        s = s + gi[:, 16 * k:16 * k + 16] * km[:, k:k + 1]
    s_ref[...] = s

def _t6_body(sg_ref, hv_ref, epf_ref, w5, v5, b5, e4_ref, hv5_ref):
    dis = epf_ref[:, 0:1]
    h3 = jnp.maximum(sg_ref[...] * dis + hv_ref[...], 0.0)
    e4_ref[...] = jnp.dot(h3, w5[...], preferred_element_type=jnp.float32) * dis
    hv5_ref[...] = jnp.dot(h3, v5[...], preferred_element_type=jnp.float32) + b5[...]


def _t8_body(sg_ref, hv_ref, epf_ref, h4_ref):
    dis = epf_ref[:, 0:1]
    h4_ref[...] = jnp.maximum(sg_ref[...] * dis + hv_ref[...], 0.0)


def _t9_body(gi_ref, ng_ref, s_ref):
    gi = gi_ref[...]
    xe6 = ng_ref[:, 20:36]
    km = ng_ref[:, 36:44]
    s = jnp.zeros((_BVS, 16), jnp.float32)
    for k in range(_CAP):
        s = s + jnp.maximum(gi[:, 16 * k:16 * k + 16] + xe6, 0.0) * km[:, k:k + 1]
    s_ref[...] = s

def _t10_body(sg_ref, h4_ref, epf_ref, eps, w1, b1, w2, b2, w3, b3, out_ref):
    mf = epf_ref[:, 1:2]
    t = (1.0 + eps[...]) * h4_ref[...] + sg_ref[...] * mf
    t = _lrelu(jnp.dot(t, w1[...], preferred_element_type=jnp.float32) + b1[...], 0.01)
    t = _lrelu(jnp.dot(t, w2[...], preferred_element_type=jnp.float32) + b2[...], 0.01)
    out_ref[...] = jnp.dot(t, w3[...], preferred_element_type=jnp.float32) + b3[...]


def _group_lists(key, n, n_e, want_counts):
    ids = jnp.arange(n_e, dtype=jnp.int32)
    ks, order = lax.sort((key, ids), num_keys=2)
    newseg = jnp.concatenate([jnp.ones((1,), jnp.bool_), ks[1:] != ks[:-1]])
    seg_start = lax.cummax(jnp.where(newseg, ids, 0))
    rank_sorted = ids - seg_start
    lst = jnp.zeros((n, _CAP), jnp.int32).at[ks, rank_sorted].set(
        order, mode='drop')
    cnt_cap = None
    if want_counts:
        qs = jnp.arange(n, dtype=key.dtype)
        off = jnp.searchsorted(ks, qs).astype(jnp.int32)
        off2 = jnp.searchsorted(ks, qs, side='right').astype(jnp.int32)
        cnt_cap = jnp.minimum(off2 - off, _CAP)
    return lst, cnt_cap, order, rank_sorted, ks

def _pad_idx(idx, mult):
    b = idx.shape[0]
    bp = ((b + mult - 1) // mult) * mult
    return jnp.pad(idx, (0, bp - b))


def kernel(x, edge_index, edge_attr, line_edge_index, params):
    del line_edge_index
    p = params
    n = x.shape[0]
    n_e = edge_attr.shape[0]
    ng = n // _BV
    eg = n_e // _BE
    sg = n // _BVS
    f32 = jnp.float32

    src = edge_index[0].astype(jnp.int32)
    dst = edge_index[1].astype(jnp.int32)

    in_list, ic, _, _, _ = _group_lists(dst, n, n_e, True)
    out_list, _, out_order, rank_os, _ = _group_lists(src, n, n_e, False)
    rank_out = jnp.zeros(n_e, jnp.int32).at[out_order].set(rank_os)
    g_idx = src * _CAP + jnp.minimum(rank_out, _CAP - 1)
    has_slot = rank_out < _CAP

    in_flat = _pad_idx(in_list.reshape(-1), _NW * 128)
    out_flat = _pad_idx(out_list.reshape(-1), _NW * 128)
    g_pad = _pad_idx(g_idx, _NW * 128)
    src_pad = _pad_idx(src, _NW * 128)

    icf16 = jnp.broadcast_to(ic.astype(f32)[:, None], (n, 16))
    ic_e = _sc_gather(icf16, src_pad)[:n_e, 0]
    m_e = has_slot & (ic_e > 0.5)
    cnt_e = jnp.where(has_slot, ic_e, 0.0)

    km = (jnp.arange(_CAP)[None, :] < ic[:, None]).astype(f32)
    epki = jnp.concatenate(
        [cnt_e[:, None], m_e.astype(f32)[:, None],
         jnp.zeros((n_e, 6), f32)], axis=1)

    att_bd = jnp.kron(jnp.eye(_CAP, dtype=f32), p['g2_att'].reshape(16, 1))
    sum_mat = jnp.tile(jnp.eye(16, dtype=f32), (_CAP, 1))

    r2 = lambda w: w.reshape(1, -1)

    node_g = pl.pallas_call(
        _p0n_body, grid=(ng,),
        in_specs=[_blk((_BV, 128)), _full((128, 64)), _full((_HEADS, 16)),
                  _full((128, 16)), _full((128, 16)), _full((1, 16)),
                  _blk((_BV, _CAP))],
        out_specs=_blk((_BV, 48)),
        out_shape=jax.ShapeDtypeStruct((n, 48), f32),
    )(x, p['g1_lin_edge'], p['g1_att_edge'], p['g2_lin_edge'], p['g6_we'],
      r2(p['g6_be']), km)

    edge1, h1d, epf = pl.pallas_call(
        _p0e_body, grid=(eg,),
        in_specs=[_blk((_BE, 16)), _full((16, 64)), _full((_HEADS, 16)),
                  _full((_HEADS, 16)), _full((1, 16)), _blk((_BE, 8))],
        out_specs=(_blk((_BE, 80)), _blk((_BE, 16)), _blk((_BE, 8))),
        out_shape=(jax.ShapeDtypeStruct((n_e, 80), f32),
                   jax.ShapeDtypeStruct((n_e, 16), f32),
                   jax.ShapeDtypeStruct((n_e, 8), f32)),
    )(edge_attr, p['g1_lin'], p['g1_att_src'], p['g1_att_dst'],
      r2(p['g1_bias']), epki)

    gi1 = _sc_gather(edge1, in_flat).reshape(-1, _CAP * 80)
    go1 = _sc_gather(edge1, out_flat)
    r1 = pl.pallas_call(
        _t1_body, grid=(sg,),
        in_specs=[_blk((_BVS, _CAP * 80)), _blk((_BVS * _CAP, 80)),
                  _blk((_BVS, 48)), _full((1, 16))],
        out_specs=_blk((_BVS * _CAP, 16)),
        out_shape=jax.ShapeDtypeStruct((n * _CAP, 16), f32),
    )(gi1, go1, node_g, r2(p['g1_bias']))

    rg1 = _sc_gather(r1, g_pad)
    edge2, h2d = pl.pallas_call(
        _t2_body, grid=(eg,),
        in_specs=[_blk((_BE, 16)), _blk((_BE, 16)), _blk((_BE, 8)),
                  _full((16, 16)), _full((1, 16)), _full((16, 16)),
                  _full((1, 16)), _full((1, 16))],
        out_specs=(_blk((_BE, 32)), _blk((_BE, 16))),
        out_shape=(jax.ShapeDtypeStruct((n_e, 32), f32),
                   jax.ShapeDtypeStruct((n_e, 16), f32)),
    )(rg1, h1d, epf, p['g2_wl'], r2(p['g2_bl']), p['g2_wr'], r2(p['g2_br']),
      r2(p['g2_bias']))

    gi2 = _sc_gather(edge2, in_flat).reshape(-1, _CAP * 32)
    go2 = _sc_gather(edge2, out_flat)
    r2a = pl.pallas_call(
        _t3_body, grid=(sg,),
        in_specs=[_blk((_BVS, _CAP * 32)), _blk((_BVS * _CAP, 32)),
                  _blk((_BVS, 48)), _full((1, 16)), _full((128, 8)),
                  _full((128, 16)), _full((1, 16))],
        out_specs=_blk((_BVS * _CAP, 16)),
        out_shape=jax.ShapeDtypeStruct((n * _CAP, 16), f32),
    )(gi2, go2, node_g, r2(p['g2_att']), att_bd, sum_mat, r2(p['g2_bias']))

    rg2 = _sc_gather(r2a, g_pad)
    edge3, hv4 = pl.pallas_call(
        _t4_body, grid=(eg,),
        in_specs=[_blk((_BE, 16)), _blk((_BE, 16)), _blk((_BE, 8)),
                  _full((16, 16)), _full((16, 16)), _full((1, 16))],
        out_specs=(_blk((_BE, 16)), _blk((_BE, 16))),
        out_shape=(jax.ShapeDtypeStruct((n_e, 16), f32),
                   jax.ShapeDtypeStruct((n_e, 16), f32)),
    )(rg2, h2d, epf, p['a4_w'], p['a4_v'], r2(p['a4_b']))

    gi3 = _sc_gather(edge3, in_flat).reshape(-1, _CAP * 16)
    s1 = pl.pallas_call(
        _t5_body, grid=(sg,),
        in_specs=[_blk((_BVS, _CAP * 16)), _blk((_BVS, 48))],
        out_specs=_blk((_BVS, 16)),
        out_shape=jax.ShapeDtypeStruct((n, 16), f32),
    )(gi3, node_g)
    sg1 = _sc_gather(s1, src_pad)
    edge4, hv5 = pl.pallas_call(
        _t6_body, grid=(eg,),
        in_specs=[_blk((_BE, 16)), _blk((_BE, 16)), _blk((_BE, 8)),
                  _full((16, 16)), _full((16, 16)), _full((1, 16))],
        out_specs=(_blk((_BE, 16)), _blk((_BE, 16))),
        out_shape=(jax.ShapeDtypeStruct((n_e, 16), f32),
                   jax.ShapeDtypeStruct((n_e, 16), f32)),
    )(sg1, hv4, epf, p['a5_w'], p['a5_v'], r2(p['a5_b']))

    gi4 = _sc_gather(edge4, in_flat).reshape(-1, _CAP * 16)
    s2 = pl.pallas_call(
        _t5_body, grid=(sg,),
        in_specs=[_blk((_BVS, _CAP * 16)), _blk((_BVS, 48))],
        out_specs=_blk((_BVS, 16)),
        out_shape=jax.ShapeDtypeStruct((n, 16), f32),
    )(gi4, node_g)
    sg2 = _sc_gather(s2, src_pad)
    h4 = pl.pallas_call(
        _t8_body, grid=(eg,),
        in_specs=[_blk((_BE, 16)), _blk((_BE, 16)), _blk((_BE, 8))],
        out_specs=_blk((_BE, 16)),
        out_shape=jax.ShapeDtypeStruct((n_e, 16), f32),
    )(sg2, hv5, epf)

    gi5 = _sc_gather(h4, in_flat).reshape(-1, _CAP * 16)
    s6 = pl.pallas_call(
        _t9_body, grid=(sg,),
        in_specs=[_blk((_BVS, _CAP * 16)), _blk((_BVS, 48))],
        out_specs=_blk((_BVS, 16)),
        out_shape=jax.ShapeDtypeStruct((n, 16), f32),
    )(gi5, node_g)
    sg6 = _sc_gather(s6, src_pad)
    new_ea = pl.pallas_call(
        _t10_body, grid=(eg,),
        in_specs=[_blk((_BE, 16)), _blk((_BE, 16)), _blk((_BE, 8)),
                  _full((1, 1)), _full((16, 32)), _full((1, 32)),
                  _full((32, 16)), _full((1, 16)), _full((16, 16)),
                  _full((1, 16))],
        out_specs=_blk((_BE, 16)),
        out_shape=jax.ShapeDtypeStruct((n_e, 16), f32),
    )(sg6, h4, epf, p['g6_eps'].reshape(1, 1), p['g6_m_w1'], r2(p['g6_m_b1']),
      p['g6_m_w2'], r2(p['g6_m_b2']), p['g6_m_w3'], r2(p['g6_m_b3']))

    return new_ea
